# baseline (speedup 1.0000x reference)
import functools

import jax
import jax.numpy as jnp
from jax import lax
from jax.experimental import pallas as pl
from jax.experimental.pallas import tpu as pltpu
from jax.experimental.pallas import tpu_sc as plsc

N = 10000
G = 256
H = 64
EMB = 192
E = 320000
D_IN = 128

NCORE = 2
NSUB = 16
NW = NCORE * NSUB
CHUNK = 128
EPT = 10240
EPAD = EPT * NW
NPAD = 10240
RPT = NPAD // NSUB

LOG2 = 0.6931471805599453


NCHUNK = EPT // CHUNK
QCH = 16
NQ = NCHUNK // QCH
NBUF = 5
NGRP = QCH // NBUF


def _make_sc_scatter():
    mesh = plsc.VectorSubcoreMesh(core_axis_name="c", subcore_axis_name="s")

    @functools.partial(
        pl.kernel,
        out_type=jax.ShapeDtypeStruct((NCORE * NPAD, H), jnp.float32),
        mesh=mesh,
        scratch_types=[
            pltpu.VMEM((QCH, CHUNK), jnp.int32),
            pltpu.VMEM((QCH, CHUNK), jnp.int32),
            pltpu.VMEM((NBUF, CHUNK, H), jnp.float32),
            pltpu.VMEM_SHARED((NPAD, H), jnp.float32),
            pltpu.VMEM_SHARED((N, H), jnp.float32),
        ] + [pltpu.SemaphoreType.DMA] * (2 * NBUF),
        compiler_params=pltpu.CompilerParams(use_tc_tiling_on_sc=False),
    )
    def sc_scatter(u_hbm, src2d_hbm, dst2d_hbm, out_hbm,
                   sidx, didx, rows, acc, u_sp, *sems):
        c = lax.axis_index("c")
        s = lax.axis_index("s")
        wid = c * NSUB + s
        r0 = s * RPT

        def zero_row(i, carry):
            for j in range(H // 16):
                rows[0, i, pl.ds(j * 16, 16)] = jnp.zeros((16,), jnp.float32)
            return carry

        lax.fori_loop(0, CHUNK, zero_row, 0)
        for k in range(RPT // CHUNK):
            pltpu.sync_copy(rows.at[0], acc.at[pl.ds(r0 + k * CHUNK, CHUNK)])
        u0 = s * 640

        @pl.when(s < 15)
        def _stage_full():
            pltpu.sync_copy(u_hbm.at[pl.ds(u0, 640)],
                            u_sp.at[pl.ds(u0, 640)])

        @pl.when(s == 15)
        def _stage_tail():
            pltpu.sync_copy(u_hbm.at[pl.ds(9600, 400)],
                            u_sp.at[pl.ds(9600, 400)])
        plsc.subcore_barrier()

        DEPTH = 4

        def quarter_body(q, carry):
            qb = wid * NCHUNK + q * QCH
            pltpu.sync_copy(src2d_hbm.at[pl.ds(qb, QCH)], sidx)
            pltpu.sync_copy(dst2d_hbm.at[pl.ds(qb, QCH)], didx)

            gh = [None] * QCH
            sh = [None] * QCH
            sc_waited = [False] * QCH

            def fire(j):
                b = j % NBUF
                gh[j] = pltpu.async_copy(
                    u_sp.at[sidx.at[j]], rows.at[b], sems[b])

            for j in range(DEPTH):
                fire(j)
            for j in range(QCH):
                gh[j].wait()
                if j + DEPTH < QCH:
                    p = j + DEPTH - NBUF
                    if p >= 0 and not sc_waited[p]:
                        sh[p].wait()
                        sc_waited[p] = True
                    fire(j + DEPTH)
                b = j % NBUF
                sh[j] = pltpu.async_copy(rows.at[b], acc.at[didx.at[j]],
                                         sems[NBUF + b], add=True)
            for j in range(QCH):
                if not sc_waited[j]:
                    sh[j].wait()
            return carry

        lax.fori_loop(0, NQ, quarter_body, 0)
        plsc.subcore_barrier()
        pltpu.sync_copy(acc.at[pl.ds(r0, RPT)],
                        out_hbm.at[pl.ds(c * NPAD + r0, RPT)])

    return sc_scatter


_sc_scatter_cache = []


def _scatter_parts(u, src_p, dst_p):
    if not _sc_scatter_cache:
        _sc_scatter_cache.append(_make_sc_scatter())
    return _sc_scatter_cache[0](u, src_p, dst_p)


def _mm(x, w):
    def body(x_ref, w_ref, o_ref):
        o_ref[...] = jnp.dot(x_ref[...], w_ref[...],
                             preferred_element_type=jnp.float32)

    return pl.pallas_call(
        body,
        out_shape=jax.ShapeDtypeStruct((x.shape[0], w.shape[1]), jnp.float32),
    )(x, w)


def _layer_post(u, parts, b1, w2, b2, bn_s, bn_b, batch2d, w_next):
    has_next = w_next is not None

    def body(u_ref, parts_ref, b1_ref, w2_ref, b2_ref, s_ref, bi_ref,
             bt_ref, *rest):
        if has_next:
            wn_ref, z_ref, y_ref, un_ref = rest
        else:
            z_ref, y_ref = rest
        pa = parts_ref[pl.ds(0, N), :]
        pb = parts_ref[pl.ds(NPAD, N), :]
        z = u_ref[...] + pa + pb + b1_ref[...]
        z = jnp.maximum(z, 0.0)
        z = jnp.dot(z, w2_ref[...], preferred_element_type=jnp.float32) + b2_ref[...]
        z = jnp.maximum(z, 0.0)
        m = jnp.mean(z, axis=0, keepdims=True)
        v = jnp.mean((z - m) ** 2, axis=0, keepdims=True)
        z = s_ref[...] * (z - m) / jnp.sqrt(v + 1e-5) + bi_ref[...]
        z_ref[...] = z
        onehot = (bt_ref[...] == lax.broadcasted_iota(jnp.int32, (N, G), 1)
                  ).astype(jnp.float32)
        y_ref[...] = lax.dot_general(onehot, z, (((0,), (0,)), ((), ())),
                                     preferred_element_type=jnp.float32)
        if has_next:
            un_ref[...] = jnp.dot(z, wn_ref[...],
                                  preferred_element_type=jnp.float32)

    outs = [jax.ShapeDtypeStruct((N, H), jnp.float32),
            jax.ShapeDtypeStruct((G, H), jnp.float32)]
    args = [u, parts, b1, w2, b2, bn_s, bn_b, batch2d]
    if has_next:
        outs.append(jax.ShapeDtypeStruct((N, H), jnp.float32))
        args.append(w_next)
    return pl.pallas_call(
        body, out_shape=outs,
    )(*args)


NB = 2000
NBLK = N // NB


def _loss_sums(z0, z1, z2, batch2d, y,
               gw0, gb0, gw1, gb1, gw2, gb2, gws, gbs,
               lw0, lb0, lw1, lb1, lw2, lb2, lws, lbs):
    def body(z0_ref, z1_ref, z2_ref, bt_ref, y_ref,
             gw0r, gb0r, gw1r, gb1r, gw2r, gb2r, gwsr, gbsr,
             lw0r, lb0r, lw1r, lb1r, lw2r, lb2r, lwsr, lbsr,
             pos_ref, neg_ref, g_ref):
        i = pl.program_id(0)

        @pl.when(i == 0)
        def _g():
            yv = y_ref[...]
            h = yv
            for wr, br in ((gw0r, gb0r), (gw1r, gb1r), (gw2r, gb2r)):
                h = jnp.maximum(
                    jnp.dot(h, wr[...], preferred_element_type=jnp.float32)
                    + br[...], 0.0)
            g_ref[...] = h + jnp.dot(yv, gwsr[...],
                                     preferred_element_type=jnp.float32) + gbsr[...]
            pos_ref[...] = jnp.zeros((1, 1), jnp.float32)
            neg_ref[...] = jnp.zeros((1, 1), jnp.float32)

        zb = (z0_ref[...], z1_ref[...], z2_ref[...])

        def split_mm(wr):
            return sum(jnp.dot(zb[k], wr[pl.ds(k * H, H), :],
                               preferred_element_type=jnp.float32)
                       for k in range(3))

        h = jnp.maximum(split_mm(lw0r) + lb0r[...], 0.0)
        for wr, br in ((lw1r, lb1r), (lw2r, lb2r)):
            h = jnp.maximum(
                jnp.dot(h, wr[...], preferred_element_type=jnp.float32)
                + br[...], 0.0)
        l_enc = h + split_mm(lwsr) + lbsr[...]
        res = lax.dot_general(l_enc, g_ref[...], (((1,), (1,)), ((), ())),
                              preferred_element_type=jnp.float32)
        posm = (bt_ref[...] == lax.broadcasted_iota(jnp.int32, (NB, G), 1)
                ).astype(jnp.float32)
        sp = jnp.maximum(-res, 0.0) + jnp.log(1.0 + jnp.exp(-jnp.abs(res)))
        pos_part = jnp.sum(posm * (LOG2 - sp))
        neg_part = jnp.sum((1.0 - posm) * (sp + res - LOG2))
        pos_ref[...] = pos_ref[...] + pos_part
        neg_ref[...] = neg_ref[...] + neg_part

    full = lambda shape: pl.BlockSpec(shape, lambda i: (0, 0))
    blk = pl.BlockSpec((NB, H), lambda i: (i, 0))
    return pl.pallas_call(
        body,
        grid=(NBLK,),
        in_specs=[
            blk, blk, blk,
            pl.BlockSpec((NB, 1), lambda i: (i, 0)),
            full((G, EMB)),
            full((EMB, EMB)), full((1, EMB)),
            full((EMB, EMB)), full((1, EMB)),
            full((EMB, EMB)), full((1, EMB)),
            full((EMB, EMB)), full((1, EMB)),
            full((EMB, EMB)), full((1, EMB)),
            full((EMB, EMB)), full((1, EMB)),
            full((EMB, EMB)), full((1, EMB)),
            full((EMB, EMB)), full((1, EMB)),
        ],
        out_specs=[pl.BlockSpec((1, 1), lambda i: (0, 0)),
                   pl.BlockSpec((1, 1), lambda i: (0, 0))],
        out_shape=[jax.ShapeDtypeStruct((1, 1), jnp.float32),
                   jax.ShapeDtypeStruct((1, 1), jnp.float32)],
        scratch_shapes=[pltpu.VMEM((G, EMB), jnp.float32)],
    )(z0, z1, z2, batch2d, y,
      gw0, gb0, gw1, gb1, gw2, gb2, gws, gbs,
      lw0, lb0, lw1, lb1, lw2, lb2, lws, lbs)


def kernel(x, label, edge_index, batch, num_graphs,
           conv0_W1, conv0_b1, conv0_W2, conv0_b2, bn0_scale, bn0_bias,
           conv1_W1, conv1_b1, conv1_W2, conv1_b2, bn1_scale, bn1_bias,
           conv2_W1, conv2_b1, conv2_W2, conv2_b2, bn2_scale, bn2_bias,
           ld_W0, ld_b0, ld_W1, ld_b1, ld_W2, ld_b2, ld_Ws, ld_bs,
           gd_W0, gd_b0, gd_W1, gd_b1, gd_W2, gd_b2, gd_Ws, gd_bs):
    src = edge_index[0]
    dst = edge_index[1]
    epad = EPAD - E
    src_p = jnp.concatenate([src, jnp.zeros((epad,), jnp.int32)]
                            ).reshape(NW * NCHUNK, CHUNK)
    dst_p = jnp.concatenate([dst, jnp.full((epad,), N, jnp.int32)]
                            ).reshape(NW * NCHUNK, CHUNK)
    batch2d = batch.reshape(N, 1)

    row2 = lambda a: a.reshape(1, -1)

    u0 = _mm(x, conv0_W1)
    parts = _scatter_parts(u0, src_p, dst_p)
    z0, y0, u1 = _layer_post(u0, parts,
                             row2(conv0_b1), conv0_W2, row2(conv0_b2),
                             row2(bn0_scale), row2(bn0_bias), batch2d, conv1_W1)
    parts = _scatter_parts(u1, src_p, dst_p)
    z1, y1, u2 = _layer_post(u1, parts,
                             row2(conv1_b1), conv1_W2, row2(conv1_b2),
                             row2(bn1_scale), row2(bn1_bias), batch2d, conv2_W1)
    parts = _scatter_parts(u2, src_p, dst_p)
    z2, y2 = _layer_post(u2, parts,
                         row2(conv2_b1), conv2_W2, row2(conv2_b2),
                         row2(bn2_scale), row2(bn2_bias), batch2d, None)

    y = jnp.concatenate([y0, y1, y2], axis=1)
    pos, neg = _loss_sums(z0, z1, z2, batch2d, y,
                          gd_W0, row2(gd_b0), gd_W1, row2(gd_b1),
                          gd_W2, row2(gd_b2), gd_Ws, row2(gd_bs),
                          ld_W0, row2(ld_b0), ld_W1, row2(ld_b1),
                          ld_W2, row2(ld_b2), ld_Ws, row2(ld_bs))
    e_pos = pos[0, 0] / N
    e_neg = neg[0, 0] / (N * (num_graphs - 1))
    return e_neg - e_pos


# speedup vs baseline: 1.0093x; 1.0093x over previous
import functools

import jax
import jax.numpy as jnp
from jax import lax
from jax.experimental import pallas as pl
from jax.experimental.pallas import tpu as pltpu
from jax.experimental.pallas import tpu_sc as plsc

N = 10000
G = 256
H = 64
EMB = 192
E = 320000
D_IN = 128

NCORE = 2
NSUB = 16
NW = NCORE * NSUB
CHUNK = 128
EPT = 10240
EPAD = EPT * NW
NPAD = 10240
RPT = NPAD // NSUB

LOG2 = 0.6931471805599453


NCHUNK = EPT // CHUNK
QCH = 20
NQ = NCHUNK // QCH
NBUF = 4
NGRP = QCH // NBUF


def _make_sc_scatter():
    mesh = plsc.VectorSubcoreMesh(core_axis_name="c", subcore_axis_name="s")

    @functools.partial(
        pl.kernel,
        out_type=jax.ShapeDtypeStruct((NCORE * NPAD, H), jnp.float32),
        mesh=mesh,
        scratch_types=[
            pltpu.VMEM((QCH, CHUNK), jnp.int32),
            pltpu.VMEM((QCH, CHUNK), jnp.int32),
            pltpu.VMEM((NBUF, CHUNK, H), jnp.float32),
            pltpu.VMEM_SHARED((NPAD, H), jnp.float32),
            pltpu.VMEM_SHARED((N, H), jnp.float32),
        ] + [pltpu.SemaphoreType.DMA] * (2 * NBUF),
        compiler_params=pltpu.CompilerParams(use_tc_tiling_on_sc=False),
    )
    def sc_scatter(u_hbm, src2d_hbm, dst2d_hbm, out_hbm,
                   sidx, didx, rows, acc, u_sp, *sems):
        c = lax.axis_index("c")
        s = lax.axis_index("s")
        wid = c * NSUB + s
        r0 = s * RPT

        def zero_row(i, carry):
            for j in range(H // 16):
                rows[0, i, pl.ds(j * 16, 16)] = jnp.zeros((16,), jnp.float32)
            return carry

        lax.fori_loop(0, CHUNK, zero_row, 0)
        for k in range(RPT // CHUNK):
            pltpu.sync_copy(rows.at[0], acc.at[pl.ds(r0 + k * CHUNK, CHUNK)])
        u0 = s * 640

        @pl.when(s < 15)
        def _stage_full():
            pltpu.sync_copy(u_hbm.at[pl.ds(u0, 640)],
                            u_sp.at[pl.ds(u0, 640)])

        @pl.when(s == 15)
        def _stage_tail():
            pltpu.sync_copy(u_hbm.at[pl.ds(9600, 400)],
                            u_sp.at[pl.ds(9600, 400)])
        plsc.subcore_barrier()

        DEPTH = 3

        def quarter_body(q, carry):
            qb = wid * NCHUNK + q * QCH
            pltpu.sync_copy(src2d_hbm.at[pl.ds(qb, QCH)], sidx)
            pltpu.sync_copy(dst2d_hbm.at[pl.ds(qb, QCH)], didx)

            gh = [None] * QCH
            sh = [None] * QCH
            sc_waited = [False] * QCH

            def fire(j):
                b = j % NBUF
                gh[j] = pltpu.async_copy(
                    u_sp.at[sidx.at[j]], rows.at[b], sems[b])

            for j in range(DEPTH):
                fire(j)
            for j in range(QCH):
                gh[j].wait()
                if j + DEPTH < QCH:
                    p = j + DEPTH - NBUF
                    if p >= 0 and not sc_waited[p]:
                        sh[p].wait()
                        sc_waited[p] = True
                    fire(j + DEPTH)
                b = j % NBUF
                sh[j] = pltpu.async_copy(rows.at[b], acc.at[didx.at[j]],
                                         sems[NBUF + b], add=True)
            for j in range(QCH):
                if not sc_waited[j]:
                    sh[j].wait()
            return carry

        lax.fori_loop(0, NQ, quarter_body, 0)
        plsc.subcore_barrier()
        pltpu.sync_copy(acc.at[pl.ds(r0, RPT)],
                        out_hbm.at[pl.ds(c * NPAD + r0, RPT)])

    return sc_scatter


_sc_scatter_cache = []


def _scatter_parts(u, src_p, dst_p):
    if not _sc_scatter_cache:
        _sc_scatter_cache.append(_make_sc_scatter())
    return _sc_scatter_cache[0](u, src_p, dst_p)


def _mm(x, w):
    def body(x_ref, w_ref, o_ref):
        o_ref[...] = jnp.dot(x_ref[...], w_ref[...],
                             preferred_element_type=jnp.float32)

    return pl.pallas_call(
        body,
        out_shape=jax.ShapeDtypeStruct((x.shape[0], w.shape[1]), jnp.float32),
    )(x, w)


def _layer_post(u, parts, b1, w2, b2, bn_s, bn_b, batch2d, w_next):
    has_next = w_next is not None

    def body(u_ref, parts_ref, b1_ref, w2_ref, b2_ref, s_ref, bi_ref,
             bt_ref, *rest):
        if has_next:
            wn_ref, z_ref, y_ref, un_ref = rest
        else:
            z_ref, y_ref = rest
        pa = parts_ref[pl.ds(0, N), :]
        pb = parts_ref[pl.ds(NPAD, N), :]
        z = u_ref[...] + pa + pb + b1_ref[...]
        z = jnp.maximum(z, 0.0)
        z = jnp.dot(z, w2_ref[...], preferred_element_type=jnp.float32) + b2_ref[...]
        z = jnp.maximum(z, 0.0)
        m = jnp.mean(z, axis=0, keepdims=True)
        v = jnp.mean((z - m) ** 2, axis=0, keepdims=True)
        z = s_ref[...] * (z - m) / jnp.sqrt(v + 1e-5) + bi_ref[...]
        z_ref[...] = z
        onehot = (bt_ref[...] == lax.broadcasted_iota(jnp.int32, (N, G), 1)
                  ).astype(jnp.float32)
        y_ref[...] = lax.dot_general(onehot, z, (((0,), (0,)), ((), ())),
                                     preferred_element_type=jnp.float32)
        if has_next:
            un_ref[...] = jnp.dot(z, wn_ref[...],
                                  preferred_element_type=jnp.float32)

    outs = [jax.ShapeDtypeStruct((N, H), jnp.float32),
            jax.ShapeDtypeStruct((G, H), jnp.float32)]
    args = [u, parts, b1, w2, b2, bn_s, bn_b, batch2d]
    if has_next:
        outs.append(jax.ShapeDtypeStruct((N, H), jnp.float32))
        args.append(w_next)
    return pl.pallas_call(
        body, out_shape=outs,
    )(*args)


NB = 2000
NBLK = N // NB


def _loss_sums(z0, z1, z2, batch2d, y,
               gw0, gb0, gw1, gb1, gw2, gb2, gws, gbs,
               lw0, lb0, lw1, lb1, lw2, lb2, lws, lbs):
    def body(z0_ref, z1_ref, z2_ref, bt_ref, y_ref,
             gw0r, gb0r, gw1r, gb1r, gw2r, gb2r, gwsr, gbsr,
             lw0r, lb0r, lw1r, lb1r, lw2r, lb2r, lwsr, lbsr,
             pos_ref, neg_ref, g_ref):
        i = pl.program_id(0)

        @pl.when(i == 0)
        def _g():
            yv = y_ref[...]
            h = yv
            for wr, br in ((gw0r, gb0r), (gw1r, gb1r), (gw2r, gb2r)):
                h = jnp.maximum(
                    jnp.dot(h, wr[...], preferred_element_type=jnp.float32)
                    + br[...], 0.0)
            g_ref[...] = h + jnp.dot(yv, gwsr[...],
                                     preferred_element_type=jnp.float32) + gbsr[...]
            pos_ref[...] = jnp.zeros((1, 1), jnp.float32)
            neg_ref[...] = jnp.zeros((1, 1), jnp.float32)

        zb = (z0_ref[...], z1_ref[...], z2_ref[...])

        def split_mm(wr):
            return sum(jnp.dot(zb[k], wr[pl.ds(k * H, H), :],
                               preferred_element_type=jnp.float32)
                       for k in range(3))

        h = jnp.maximum(split_mm(lw0r) + lb0r[...], 0.0)
        for wr, br in ((lw1r, lb1r), (lw2r, lb2r)):
            h = jnp.maximum(
                jnp.dot(h, wr[...], preferred_element_type=jnp.float32)
                + br[...], 0.0)
        l_enc = h + split_mm(lwsr) + lbsr[...]
        res = lax.dot_general(l_enc, g_ref[...], (((1,), (1,)), ((), ())),
                              preferred_element_type=jnp.float32)
        posm = (bt_ref[...] == lax.broadcasted_iota(jnp.int32, (NB, G), 1)
                ).astype(jnp.float32)
        sp = jnp.maximum(-res, 0.0) + jnp.log(1.0 + jnp.exp(-jnp.abs(res)))
        pos_part = jnp.sum(posm * (LOG2 - sp))
        neg_part = jnp.sum((1.0 - posm) * (sp + res - LOG2))
        pos_ref[...] = pos_ref[...] + pos_part
        neg_ref[...] = neg_ref[...] + neg_part

    full = lambda shape: pl.BlockSpec(shape, lambda i: (0, 0))
    blk = pl.BlockSpec((NB, H), lambda i: (i, 0))
    return pl.pallas_call(
        body,
        grid=(NBLK,),
        in_specs=[
            blk, blk, blk,
            pl.BlockSpec((NB, 1), lambda i: (i, 0)),
            full((G, EMB)),
            full((EMB, EMB)), full((1, EMB)),
            full((EMB, EMB)), full((1, EMB)),
            full((EMB, EMB)), full((1, EMB)),
            full((EMB, EMB)), full((1, EMB)),
            full((EMB, EMB)), full((1, EMB)),
            full((EMB, EMB)), full((1, EMB)),
            full((EMB, EMB)), full((1, EMB)),
            full((EMB, EMB)), full((1, EMB)),
        ],
        out_specs=[pl.BlockSpec((1, 1), lambda i: (0, 0)),
                   pl.BlockSpec((1, 1), lambda i: (0, 0))],
        out_shape=[jax.ShapeDtypeStruct((1, 1), jnp.float32),
                   jax.ShapeDtypeStruct((1, 1), jnp.float32)],
        scratch_shapes=[pltpu.VMEM((G, EMB), jnp.float32)],
    )(z0, z1, z2, batch2d, y,
      gw0, gb0, gw1, gb1, gw2, gb2, gws, gbs,
      lw0, lb0, lw1, lb1, lw2, lb2, lws, lbs)


def kernel(x, label, edge_index, batch, num_graphs,
           conv0_W1, conv0_b1, conv0_W2, conv0_b2, bn0_scale, bn0_bias,
           conv1_W1, conv1_b1, conv1_W2, conv1_b2, bn1_scale, bn1_bias,
           conv2_W1, conv2_b1, conv2_W2, conv2_b2, bn2_scale, bn2_bias,
           ld_W0, ld_b0, ld_W1, ld_b1, ld_W2, ld_b2, ld_Ws, ld_bs,
           gd_W0, gd_b0, gd_W1, gd_b1, gd_W2, gd_b2, gd_Ws, gd_bs):
    src = edge_index[0]
    dst = edge_index[1]
    epad = EPAD - E
    src_p = jnp.concatenate([src, jnp.zeros((epad,), jnp.int32)]
                            ).reshape(NW * NCHUNK, CHUNK)
    dst_p = jnp.concatenate([dst, jnp.full((epad,), N, jnp.int32)]
                            ).reshape(NW * NCHUNK, CHUNK)
    batch2d = batch.reshape(N, 1)

    row2 = lambda a: a.reshape(1, -1)

    u0 = _mm(x, conv0_W1)
    parts = _scatter_parts(u0, src_p, dst_p)
    z0, y0, u1 = _layer_post(u0, parts,
                             row2(conv0_b1), conv0_W2, row2(conv0_b2),
                             row2(bn0_scale), row2(bn0_bias), batch2d, conv1_W1)
    parts = _scatter_parts(u1, src_p, dst_p)
    z1, y1, u2 = _layer_post(u1, parts,
                             row2(conv1_b1), conv1_W2, row2(conv1_b2),
                             row2(bn1_scale), row2(bn1_bias), batch2d, conv2_W1)
    parts = _scatter_parts(u2, src_p, dst_p)
    z2, y2 = _layer_post(u2, parts,
                         row2(conv2_b1), conv2_W2, row2(conv2_b2),
                         row2(bn2_scale), row2(bn2_bias), batch2d, None)

    y = jnp.concatenate([y0, y1, y2], axis=1)
    pos, neg = _loss_sums(z0, z1, z2, batch2d, y,
                          gd_W0, row2(gd_b0), gd_W1, row2(gd_b1),
                          gd_W2, row2(gd_b2), gd_Ws, row2(gd_bs),
                          ld_W0, row2(ld_b0), ld_W1, row2(ld_b1),
                          ld_W2, row2(ld_b2), ld_Ws, row2(ld_bs))
    e_pos = pos[0, 0] / N
    e_neg = neg[0, 0] / (N * (num_graphs - 1))
    return e_neg - e_pos


# speedup vs baseline: 1.0208x; 1.0113x over previous
import functools

import jax
import jax.numpy as jnp
from jax import lax
from jax.experimental import pallas as pl
from jax.experimental.pallas import tpu as pltpu
from jax.experimental.pallas import tpu_sc as plsc

N = 10000
G = 256
H = 64
EMB = 192
E = 320000
D_IN = 128

NCORE = 2
NSUB = 16
NW = NCORE * NSUB
CHUNK = 128
EPT = 10240
EPAD = EPT * NW
NPAD = 10240
RPT = NPAD // NSUB

LOG2 = 0.6931471805599453


NCHUNK = EPT // CHUNK
QCH = 20
NQ = NCHUNK // QCH
NBUF = 4
NGRP = QCH // NBUF


def _make_sc_scatter():
    mesh = plsc.VectorSubcoreMesh(core_axis_name="c", subcore_axis_name="s")

    @functools.partial(
        pl.kernel,
        out_type=jax.ShapeDtypeStruct((NCORE * NPAD, H), jnp.float32),
        mesh=mesh,
        scratch_types=[
            pltpu.VMEM((QCH, CHUNK), jnp.int32),
            pltpu.VMEM((QCH, CHUNK), jnp.int32),
            pltpu.VMEM((NBUF, CHUNK, H), jnp.float32),
            pltpu.VMEM_SHARED((NPAD, H), jnp.float32),
            pltpu.VMEM_SHARED((N, H), jnp.float32),
        ] + [pltpu.SemaphoreType.DMA] * (2 * NBUF),
        compiler_params=pltpu.CompilerParams(use_tc_tiling_on_sc=False),
    )
    def sc_scatter(u_hbm, src2d_hbm, dst2d_hbm, out_hbm,
                   sidx, didx, rows, acc, u_sp, *sems):
        c = lax.axis_index("c")
        s = lax.axis_index("s")
        wid = c * NSUB + s
        r0 = s * RPT

        def zero_row(i, carry):
            for j in range(H // 16):
                rows[0, i, pl.ds(j * 16, 16)] = jnp.zeros((16,), jnp.float32)
            return carry

        lax.fori_loop(0, CHUNK, zero_row, 0)
        for k in range(RPT // CHUNK):
            pltpu.sync_copy(rows.at[0], acc.at[pl.ds(r0 + k * CHUNK, CHUNK)])
        u0 = s * 640

        @pl.when(s < 15)
        def _stage_full():
            pltpu.sync_copy(u_hbm.at[pl.ds(u0, 640)],
                            u_sp.at[pl.ds(u0, 640)])

        @pl.when(s == 15)
        def _stage_tail():
            pltpu.sync_copy(u_hbm.at[pl.ds(9600, 400)],
                            u_sp.at[pl.ds(9600, 400)])
        plsc.subcore_barrier()

        DEPTH = 3

        def quarter_body(q, carry):
            qb = wid * NCHUNK + q * QCH
            pltpu.sync_copy(src2d_hbm.at[pl.ds(qb, QCH)], sidx)
            pltpu.sync_copy(dst2d_hbm.at[pl.ds(qb, QCH)], didx)

            gh = [None] * QCH
            sh = [None] * QCH
            sc_waited = [False] * QCH

            def fire(j):
                b = j % NBUF
                gh[j] = pltpu.async_copy(
                    u_sp.at[sidx.at[j]], rows.at[b], sems[b])

            for j in range(DEPTH):
                fire(j)
            for j in range(QCH):
                gh[j].wait()
                if j + DEPTH < QCH:
                    p = j + DEPTH - NBUF
                    if p >= 0 and not sc_waited[p]:
                        sh[p].wait()
                        sc_waited[p] = True
                    fire(j + DEPTH)
                b = j % NBUF
                sh[j] = pltpu.async_copy(rows.at[b], acc.at[didx.at[j]],
                                         sems[NBUF + b], add=True)
            for j in range(QCH):
                if not sc_waited[j]:
                    sh[j].wait()
            return carry

        lax.fori_loop(0, NQ, quarter_body, 0)
        plsc.subcore_barrier()
        pltpu.sync_copy(acc.at[pl.ds(r0, RPT)],
                        out_hbm.at[pl.ds(c * NPAD + r0, RPT)])

    return sc_scatter


_sc_scatter_cache = []


def _scatter_parts(u, src_p, dst_p):
    if not _sc_scatter_cache:
        _sc_scatter_cache.append(_make_sc_scatter())
    return _sc_scatter_cache[0](u, src_p, dst_p)


def _mm(x, w):
    def body(x_ref, w_ref, o_ref):
        o_ref[...] = jnp.dot(x_ref[...], w_ref[...],
                             preferred_element_type=jnp.float32)

    return pl.pallas_call(
        body,
        out_shape=jax.ShapeDtypeStruct((x.shape[0], w.shape[1]), jnp.float32),
    )(x, w)


def _layer_post(u, parts, b1, w2, b2, bn_s, bn_b, batch2d, w_next):
    has_next = w_next is not None

    def body(u_ref, parts_ref, b1_ref, w2_ref, b2_ref, s_ref, bi_ref,
             bt_ref, *rest):
        if has_next:
            wn_ref, z_ref, un_ref = rest
        else:
            z_ref, y_ref = rest
        pa = parts_ref[pl.ds(0, N), :]
        pb = parts_ref[pl.ds(NPAD, N), :]
        z = u_ref[...] + pa + pb + b1_ref[...]
        z = jnp.maximum(z, 0.0)
        z = jnp.dot(z, w2_ref[...], preferred_element_type=jnp.float32) + b2_ref[...]
        z = jnp.maximum(z, 0.0)
        m = jnp.mean(z, axis=0, keepdims=True)
        v = jnp.mean((z - m) ** 2, axis=0, keepdims=True)
        z = s_ref[...] * (z - m) / jnp.sqrt(v + 1e-5) + bi_ref[...]
        z_ref[...] = z
        if has_next:
            un_ref[...] = jnp.dot(z, wn_ref[...],
                                  preferred_element_type=jnp.float32)
        else:
            onehot = (bt_ref[...] == lax.broadcasted_iota(jnp.int32, (N, G), 1)
                      ).astype(jnp.float32)
            y_ref[...] = lax.dot_general(onehot, z, (((0,), (0,)), ((), ())),
                                         preferred_element_type=jnp.float32)

    outs = [jax.ShapeDtypeStruct((N, H), jnp.float32)]
    args = [u, parts, b1, w2, b2, bn_s, bn_b, batch2d]
    if has_next:
        outs.append(jax.ShapeDtypeStruct((N, H), jnp.float32))
        args.append(w_next)
    else:
        outs.append(jax.ShapeDtypeStruct((G, H), jnp.float32))
    return pl.pallas_call(
        body, out_shape=outs,
    )(*args)


def _pool(z, batch2d):
    def body(z_ref, bt_ref, y_ref):
        onehot = (bt_ref[...] == lax.broadcasted_iota(jnp.int32, (N, G), 1)
                  ).astype(jnp.float32)
        y_ref[...] = lax.dot_general(onehot, z_ref[...],
                                     (((0,), (0,)), ((), ())),
                                     preferred_element_type=jnp.float32)

    return pl.pallas_call(
        body,
        out_shape=jax.ShapeDtypeStruct((G, H), jnp.float32),
    )(z, batch2d)


NB = 2000
NBLK = N // NB


def _loss_sums(z0, z1, z2, batch2d, y,
               gw0, gb0, gw1, gb1, gw2, gb2, gws, gbs,
               lw0, lb0, lw1, lb1, lw2, lb2, lws, lbs):
    def body(z0_ref, z1_ref, z2_ref, bt_ref, y_ref,
             gw0r, gb0r, gw1r, gb1r, gw2r, gb2r, gwsr, gbsr,
             lw0r, lb0r, lw1r, lb1r, lw2r, lb2r, lwsr, lbsr,
             pos_ref, neg_ref, g_ref):
        i = pl.program_id(0)

        @pl.when(i == 0)
        def _g():
            yv = y_ref[...]
            h = yv
            for wr, br in ((gw0r, gb0r), (gw1r, gb1r), (gw2r, gb2r)):
                h = jnp.maximum(
                    jnp.dot(h, wr[...], preferred_element_type=jnp.float32)
                    + br[...], 0.0)
            g_ref[...] = h + jnp.dot(yv, gwsr[...],
                                     preferred_element_type=jnp.float32) + gbsr[...]
            pos_ref[...] = jnp.zeros((1, 1), jnp.float32)
            neg_ref[...] = jnp.zeros((1, 1), jnp.float32)

        zb = (z0_ref[...], z1_ref[...], z2_ref[...])

        def split_mm(wr):
            return sum(jnp.dot(zb[k], wr[pl.ds(k * H, H), :],
                               preferred_element_type=jnp.float32)
                       for k in range(3))

        h = jnp.maximum(split_mm(lw0r) + lb0r[...], 0.0)
        for wr, br in ((lw1r, lb1r), (lw2r, lb2r)):
            h = jnp.maximum(
                jnp.dot(h, wr[...], preferred_element_type=jnp.float32)
                + br[...], 0.0)
        l_enc = h + split_mm(lwsr) + lbsr[...]
        res = lax.dot_general(l_enc, g_ref[...], (((1,), (1,)), ((), ())),
                              preferred_element_type=jnp.float32)
        posm = (bt_ref[...] == lax.broadcasted_iota(jnp.int32, (NB, G), 1)
                ).astype(jnp.float32)
        sp = jnp.maximum(-res, 0.0) + jnp.log(1.0 + jnp.exp(-jnp.abs(res)))
        pos_part = jnp.sum(posm * (LOG2 - sp))
        neg_part = jnp.sum((1.0 - posm) * (sp + res - LOG2))
        pos_ref[...] = pos_ref[...] + pos_part
        neg_ref[...] = neg_ref[...] + neg_part

    full = lambda shape: pl.BlockSpec(shape, lambda i: (0, 0))
    blk = pl.BlockSpec((NB, H), lambda i: (i, 0))
    return pl.pallas_call(
        body,
        grid=(NBLK,),
        in_specs=[
            blk, blk, blk,
            pl.BlockSpec((NB, 1), lambda i: (i, 0)),
            full((G, EMB)),
            full((EMB, EMB)), full((1, EMB)),
            full((EMB, EMB)), full((1, EMB)),
            full((EMB, EMB)), full((1, EMB)),
            full((EMB, EMB)), full((1, EMB)),
            full((EMB, EMB)), full((1, EMB)),
            full((EMB, EMB)), full((1, EMB)),
            full((EMB, EMB)), full((1, EMB)),
            full((EMB, EMB)), full((1, EMB)),
        ],
        out_specs=[pl.BlockSpec((1, 1), lambda i: (0, 0)),
                   pl.BlockSpec((1, 1), lambda i: (0, 0))],
        out_shape=[jax.ShapeDtypeStruct((1, 1), jnp.float32),
                   jax.ShapeDtypeStruct((1, 1), jnp.float32)],
        scratch_shapes=[pltpu.VMEM((G, EMB), jnp.float32)],
    )(z0, z1, z2, batch2d, y,
      gw0, gb0, gw1, gb1, gw2, gb2, gws, gbs,
      lw0, lb0, lw1, lb1, lw2, lb2, lws, lbs)


def kernel(x, label, edge_index, batch, num_graphs,
           conv0_W1, conv0_b1, conv0_W2, conv0_b2, bn0_scale, bn0_bias,
           conv1_W1, conv1_b1, conv1_W2, conv1_b2, bn1_scale, bn1_bias,
           conv2_W1, conv2_b1, conv2_W2, conv2_b2, bn2_scale, bn2_bias,
           ld_W0, ld_b0, ld_W1, ld_b1, ld_W2, ld_b2, ld_Ws, ld_bs,
           gd_W0, gd_b0, gd_W1, gd_b1, gd_W2, gd_b2, gd_Ws, gd_bs):
    src = edge_index[0]
    dst = edge_index[1]
    epad = EPAD - E
    src_p = jnp.concatenate([src, jnp.zeros((epad,), jnp.int32)]
                            ).reshape(NW * NCHUNK, CHUNK)
    dst_p = jnp.concatenate([dst, jnp.full((epad,), N, jnp.int32)]
                            ).reshape(NW * NCHUNK, CHUNK)
    batch2d = batch.reshape(N, 1)

    row2 = lambda a: a.reshape(1, -1)

    u0 = _mm(x, conv0_W1)
    parts = _scatter_parts(u0, src_p, dst_p)
    z0, u1 = _layer_post(u0, parts,
                         row2(conv0_b1), conv0_W2, row2(conv0_b2),
                         row2(bn0_scale), row2(bn0_bias), batch2d, conv1_W1)
    parts = _scatter_parts(u1, src_p, dst_p)
    y0 = _pool(z0, batch2d)
    z1, u2 = _layer_post(u1, parts,
                         row2(conv1_b1), conv1_W2, row2(conv1_b2),
                         row2(bn1_scale), row2(bn1_bias), batch2d, conv2_W1)
    parts = _scatter_parts(u2, src_p, dst_p)
    y1 = _pool(z1, batch2d)
    z2, y2 = _layer_post(u2, parts,
                         row2(conv2_b1), conv2_W2, row2(conv2_b2),
                         row2(bn2_scale), row2(bn2_bias), batch2d, None)

    y = jnp.concatenate([y0, y1, y2], axis=1)
    pos, neg = _loss_sums(z0, z1, z2, batch2d, y,
                          gd_W0, row2(gd_b0), gd_W1, row2(gd_b1),
                          gd_W2, row2(gd_b2), gd_Ws, row2(gd_bs),
                          ld_W0, row2(ld_b0), ld_W1, row2(ld_b1),
                          ld_W2, row2(ld_b2), ld_Ws, row2(ld_bs))
    e_pos = pos[0, 0] / N
    e_neg = neg[0, 0] / (N * (num_graphs - 1))
    return e_neg - e_pos
